# Initial kernel scaffold; baseline (speedup 1.0000x reference)
import jax
import jax.numpy as jnp
from jax.experimental import pallas as pl


def kernel(feature_in, embed_in, tree, zeta_g):
    raise NotImplementedError("write your pallas kernel here")


# speedup vs baseline: 9.2818x; 9.2818x over previous
import functools

import jax
import jax.numpy as jnp
from jax import lax
from jax.experimental import pallas as pl
from jax.experimental.pallas import tpu as pltpu
from jax.experimental.pallas import tpu_sc as plsc

NC = 2
NS = 16
L = 16
CHUNK = 128


def _sc_run(table, feat, srcg, tgtg, tgtl, zeros, *, bs, n, c, e_pad, n_pad):
    t_per_tile = e_pad // NS
    n_chunks = t_per_tile // CHUNK
    s_fin = n // NS

    mesh = plsc.VectorSubcoreMesh(
        core_axis_name="c", subcore_axis_name="s",
        num_cores=NC, num_subcores=NS)

    @functools.partial(
        pl.kernel,
        out_type=jax.ShapeDtypeStruct((bs * n,), jnp.float32),
        mesh=mesh,
        compiler_params=pltpu.CompilerParams(
            use_tc_tiling_on_sc=False, needs_layout_passes=False),
        scratch_types=[
            pltpu.VMEM((CHUNK,), jnp.int32),
            pltpu.VMEM((CHUNK,), jnp.int32),
            pltpu.VMEM((CHUNK,), jnp.int32),
            pltpu.VMEM((CHUNK, c), jnp.float32),
            pltpu.VMEM((CHUNK, c), jnp.float32),
            pltpu.VMEM((CHUNK,), jnp.float32),
            pltpu.VMEM((CHUNK,), jnp.float32),
            pltpu.VMEM((CHUNK,), jnp.float32),
            pltpu.VMEM((s_fin,), jnp.float32),
            pltpu.VMEM((s_fin,), jnp.float32),
            pltpu.VMEM((s_fin,), jnp.float32),
            pltpu.VMEM((s_fin,), jnp.float32),
            pltpu.VMEM_SHARED((n_pad,), jnp.float32),
            pltpu.VMEM_SHARED((n_pad,), jnp.float32),
            pltpu.SemaphoreType.DMA,
            pltpu.SemaphoreType.DMA,
            pltpu.SemaphoreType.DMA,
        ],
    )
    def run(table_h, feat_h, srcg_h, tgtg_h, tgtl_h, zeros_h, out_h,
            idxs, idxt, idxl, srows, trows, fsrc, affb, valb,
            nbuf, dbuf, fbuf, obuf, num_sh, den_sh, sem1, sem2, sem3):
        cid = lax.axis_index("c")
        sid = lax.axis_index("s")

        ilen = n_pad // NS
        ib = sid * ilen
        pltpu.sync_copy(zeros_h.at[pl.ds(ib, ilen)], num_sh.at[pl.ds(ib, ilen)])
        pltpu.sync_copy(zeros_h.at[pl.ds(ib, ilen)], den_sh.at[pl.ds(ib, ilen)])
        plsc.subcore_barrier()

        rows_g = [g * L + lax.iota(jnp.int32, L) for g in range(CHUNK // L)]
        tbase = cid * e_pad + sid * t_per_tile

        def chunk_body(k, carry):
            base = tbase + k * CHUNK
            pltpu.sync_copy(srcg_h.at[pl.ds(base, CHUNK)], idxs)
            pltpu.sync_copy(tgtg_h.at[pl.ds(base, CHUNK)], idxt)
            pltpu.sync_copy(tgtl_h.at[pl.ds(base, CHUNK)], idxl)
            d1 = pltpu.async_copy(table_h.at[idxs], srows, sem1)
            d2 = pltpu.async_copy(table_h.at[idxt], trows, sem2)
            d3 = pltpu.async_copy(feat_h.at[idxs], fsrc, sem3)
            d1.wait()
            d2.wait()
            d3.wait()
            for g in range(CHUNK // L):
                rows = rows_g[g]

                def ch_step(i, acc_col, rows=rows):
                    acc, colv = acc_col
                    for _ in range(8):
                        sv = plsc.load_gather(srows, [rows, colv])
                        tv = plsc.load_gather(trows, [rows, colv])
                        d = sv - tv
                        acc = acc + d * d
                        colv = colv + 1
                    return acc, colv

                acc0 = jnp.zeros((L,), jnp.float32)
                col0 = jnp.zeros((L,), jnp.int32)
                acc, _ = lax.fori_loop(0, c // 8, ch_step, (acc0, col0))
                aff = jnp.exp(-acc)
                fs = fsrc[pl.ds(g * L, L)]
                affb[pl.ds(g * L, L)] = aff
                valb[pl.ds(g * L, L)] = aff * fs
            pltpu.async_copy(valb, num_sh.at[idxl], sem1, add=True).wait()
            pltpu.async_copy(affb, den_sh.at[idxl], sem2, add=True).wait()
            return carry

        lax.fori_loop(0, n_chunks, chunk_body, 0)
        plsc.subcore_barrier()

        ob = sid * s_fin
        pltpu.sync_copy(num_sh.at[pl.ds(ob, s_fin)], nbuf)
        pltpu.sync_copy(den_sh.at[pl.ds(ob, s_fin)], dbuf)
        pltpu.sync_copy(feat_h.at[pl.ds(cid * n + ob, s_fin)], fbuf)

        def fin_body(i, carry):
            sl = pl.ds(i * L, L)
            obuf[sl] = (fbuf[sl] + nbuf[sl]) / (1.0 + dbuf[sl])
            return carry

        lax.fori_loop(0, s_fin // L, fin_body, 0)
        pltpu.sync_copy(obuf, out_h.at[pl.ds(cid * n + ob, s_fin)])

    return run(table, feat, srcg, tgtg, tgtl, zeros)


def kernel(feature_in, embed_in, tree, zeta_g=0.01):
    bs, c, h, w = embed_in.shape
    n = h * w
    e = tree.shape[1]
    assert bs == NC and c % 8 == 0 and n % (NS * L) == 0

    per = NS * CHUNK
    e_pad = ((e + per - 1) // per) * per
    n_pad = ((n + 1 + 127) // 128) * 128

    zeta = jnp.asarray(zeta_g, jnp.float32)
    scale = lax.rsqrt(zeta)
    table = (embed_in.reshape(bs, c, n) * scale).transpose(0, 2, 1)
    table = table.reshape(bs * n, c)
    feat = feature_in.reshape(bs * n).astype(jnp.float32)

    src = tree[:, :, 0].astype(jnp.int32)
    tgt = tree[:, :, 1].astype(jnp.int32)
    pad = e_pad - e
    off = (jnp.arange(bs, dtype=jnp.int32) * n)[:, None]
    srcg = (jnp.pad(src, ((0, 0), (0, pad))) + off).reshape(-1)
    tgtg = (jnp.pad(tgt, ((0, 0), (0, pad))) + off).reshape(-1)
    tgtl = jnp.pad(tgt, ((0, 0), (0, pad)), constant_values=n).reshape(-1)
    zeros = jnp.zeros((n_pad,), jnp.float32)

    out = _sc_run(table, feat, srcg, tgtg, tgtl, zeros,
                  bs=bs, n=n, c=c, e_pad=e_pad, n_pad=n_pad)
    return out.reshape(bs, 1, h, w)
